# baseline (speedup 1.0000x reference)
import functools

import jax
import jax.numpy as jnp
from jax import lax
from jax.experimental import pallas as pl
from jax.experimental.pallas import tpu as pltpu
from jax.experimental.pallas import tpu_sc as plsc

F = 26
V = 100000
D = 32
B = 4096
L = 20
VP = 102400
RPF = VP * D // 128

NC = 2
NS = 16
NW = NC * NS
BPW = B // NW
C = 64
S = BPW // C
ROWS = C * L
NDMA = ROWS // 128
FH = F // 2
NCHUNK = FH * S

VBLK = 4096
TPF = 25


def _tp_body(x_ref, o_ref):
    x = x_ref[0]
    xx = jnp.concatenate(
        [x[:, q * 1024:(q + 1) * 1024] for q in range(4)], axis=0
    )
    o_ref[0] = jnp.swapaxes(xx, 0, 1).astype(jnp.bfloat16)


def _tc_flatten(tabT, f0):
    return pl.pallas_call(
        _tp_body,
        grid=(FH, TPF),
        in_specs=[pl.BlockSpec((1, D, VBLK), lambda f, vb: (f0 + f, 0, vb))],
        out_specs=pl.BlockSpec(
            (1, VBLK // 4, 128), lambda f, vb: (f, vb, 0)
        ),
        out_shape=jax.ShapeDtypeStruct((FH, RPF, 128), jnp.bfloat16),
    )(tabT)


def _sc_body(idx_hbm, tab_hbm, out_hbm,
             idx_v0, idx_v1, rows_v0, rows_v1, out_v0, out_v1,
             isem0, isem1, gsem0, gsem1, osem0, osem1):
    idx_v = [idx_v0, idx_v1]
    rows_v = [rows_v0, rows_v1]
    out_v = [out_v0, out_v1]
    isem = [isem0, isem1]
    gsem = [gsem0, gsem1]
    osem = [osem0, osem1]

    wid = lax.axis_index("s") * NC + lax.axis_index("c")

    def cid_of(k):
        return (k // S) * (B // C) + wid * S + (k % S)

    def out_dst(k):
        b0 = wid * BPW + (k % S) * C
        return out_hbm.at[pl.ds(b0, C), k // S]

    def fire_idx(k, b):
        pltpu.async_copy(idx_hbm.at[cid_of(k)], idx_v[b], isem[b])

    def wait_idx(b):
        pltpu.make_async_copy(idx_hbm.at[0], idx_v[b], isem[b]).wait()

    def fire_gathers(b):
        for j in range(NDMA):
            pltpu.async_copy(
                tab_hbm.at[idx_v[b].at[j]],
                rows_v[b].at[pl.ds(j * 128, 128)],
                gsem[b],
            )

    def drain_gathers(b):
        for j in range(NDMA):
            pltpu.make_async_copy(
                tab_hbm.at[idx_v[b].at[j]],
                rows_v[b].at[pl.ds(j * 128, 128)],
                gsem[b],
            ).wait()

    def accumulate(b):
        col_e = lax.iota(jnp.int32, 16) * 2
        col_o = col_e + 1

        @pl.loop(0, C)
        def _bag(i):
            r0 = i * L
            acc_e, acc_o = plsc.unpack(
                rows_v[b][r0, :], format=plsc.PackFormat.INTERLEAVED
            )
            for l in range(1, L):
                e, o = plsc.unpack(
                    rows_v[b][r0 + l, :], format=plsc.PackFormat.INTERLEAVED
                )
                acc_e = acc_e + e
                acc_o = acc_o + o
            row = jnp.full((16,), i, dtype=jnp.int32)
            plsc.store_scatter(out_v[b], [row, col_e], acc_e)
            plsc.store_scatter(out_v[b], [row, col_o], acc_o)

    pltpu.sync_copy(idx_hbm.at[cid_of(0)], idx_v[0])
    fire_gathers(0)
    fire_idx(1, 1)

    @pl.loop(0, NCHUNK, step=2)
    def _outer(g):
        for b in range(2):
            k = g + b
            drain_gathers(b)

            @pl.when(k + 2 < NCHUNK)
            def _():
                fire_idx(k + 2, b)

            @pl.when(k + 1 < NCHUNK)
            def _():
                wait_idx(1 - b)
                fire_gathers(1 - b)

            @pl.when(k >= 2)
            def _():
                pltpu.make_async_copy(out_v[b], out_dst(k - 2), osem[b]).wait()

            accumulate(b)
            pltpu.async_copy(out_v[b], out_dst(k), osem[b])

    pltpu.make_async_copy(out_v[0], out_dst(NCHUNK - 2), osem[0]).wait()
    pltpu.make_async_copy(out_v[1], out_dst(NCHUNK - 1), osem[1]).wait()


@jax.jit
def _sc_lookup(idx_chunks, tab_flat):
    mesh = plsc.VectorSubcoreMesh(core_axis_name="c", subcore_axis_name="s")
    return pl.kernel(
        _sc_body,
        out_type=jax.ShapeDtypeStruct((B, FH, D), jnp.float32),
        mesh=mesh,
        scratch_types=[
            pltpu.VMEM((NDMA, 128), jnp.int32),
            pltpu.VMEM((NDMA, 128), jnp.int32),
            pltpu.VMEM((ROWS, D), jnp.bfloat16),
            pltpu.VMEM((ROWS, D), jnp.bfloat16),
            pltpu.VMEM((C, D), jnp.float32),
            pltpu.VMEM((C, D), jnp.float32),
            pltpu.SemaphoreType.DMA,
            pltpu.SemaphoreType.DMA,
            pltpu.SemaphoreType.DMA,
            pltpu.SemaphoreType.DMA,
            pltpu.SemaphoreType.DMA,
            pltpu.SemaphoreType.DMA,
        ],
        compiler_params=pltpu.CompilerParams(
            use_tc_tiling_on_sc=False, needs_layout_passes=False
        ),
    )(idx_chunks, tab_flat)


def kernel(indices, tables):
    idx0 = indices.astype(jnp.int32)
    p = idx0 % VBLK
    fl = jnp.arange(F, dtype=jnp.int32)[:, None, None] % FH
    idx = (
        (fl * TPF + idx0 // VBLK) * VBLK + (p % 1024) * 4 + p // 1024
    )
    idx128 = jax.lax.optimization_barrier(idx.reshape(F * B * L // 128, 128))
    idx_chunks = idx128.reshape(
        2, FH * B * L // (NDMA * 128), NDMA, 128
    )
    tabT = jnp.transpose(tables, (0, 2, 1))
    halves = []
    for h in range(2):
        tab_h = _tc_flatten(tabT, h * FH).reshape(FH * VP, D)
        halves.append(_sc_lookup(idx_chunks[h], tab_h))
    return jnp.concatenate(halves, axis=1)


# speedup vs baseline: 1.0004x; 1.0004x over previous
import functools

import jax
import jax.numpy as jnp
from jax import lax
from jax.experimental import pallas as pl
from jax.experimental.pallas import tpu as pltpu
from jax.experimental.pallas import tpu_sc as plsc

F = 26
V = 100000
D = 32
B = 4096
L = 20
VP = 102400
RPF = VP * D // 128

NC = 2
NS = 16
NW = NC * NS
BPW = B // NW
C = 64
S = BPW // C
ROWS = C * L
NDMA = ROWS // 128
FH = F // 2
NCHUNK = FH * S

VBLK = 4096
TPF = 25


def _tp_body(x_ref, o_ref):
    x = x_ref[0]
    xx = jnp.concatenate(
        [x[:, q * 1024:(q + 1) * 1024] for q in range(4)], axis=0
    )
    o_ref[0] = jnp.swapaxes(xx, 0, 1).astype(jnp.bfloat16)


def _tc_flatten(tabT, f0):
    return pl.pallas_call(
        _tp_body,
        grid=(FH, TPF),
        in_specs=[pl.BlockSpec((1, D, VBLK), lambda f, vb: (f0 + f, 0, vb))],
        out_specs=pl.BlockSpec(
            (1, VBLK // 4, 128), lambda f, vb: (f, vb, 0)
        ),
        out_shape=jax.ShapeDtypeStruct((FH, RPF, 128), jnp.bfloat16),
    )(tabT)


def _sc_body(idx_hbm, tab_hbm, out_hbm,
             idx_v0, idx_v1, rows_v0, rows_v1, out_v0, out_v1,
             isem0, isem1, gsem0, gsem1, osem0, osem1):
    idx_v = [idx_v0, idx_v1]
    rows_v = [rows_v0, rows_v1]
    out_v = [out_v0, out_v1]
    isem = [isem0, isem1]
    gsem = [gsem0, gsem1]
    osem = [osem0, osem1]

    wid = lax.axis_index("s") * NC + lax.axis_index("c")

    def cid_of(k):
        return (k // S) * (B // C) + wid * S + (k % S)

    def out_dst(k):
        b0 = wid * BPW + (k % S) * C
        return out_hbm.at[pl.ds(b0, C), k // S]

    def fire_idx(k, b):
        pltpu.async_copy(idx_hbm.at[cid_of(k)], idx_v[b], isem[b])

    def wait_idx(b):
        pltpu.make_async_copy(idx_hbm.at[0], idx_v[b], isem[b]).wait()

    def fire_gathers(b):
        for j in range(NDMA):
            pltpu.async_copy(
                tab_hbm.at[idx_v[b].at[j]],
                rows_v[b].at[pl.ds(j * 128, 128)],
                gsem[b],
            )

    def drain_gathers(b):
        for j in range(NDMA):
            pltpu.make_async_copy(
                tab_hbm.at[idx_v[b].at[j]],
                rows_v[b].at[pl.ds(j * 128, 128)],
                gsem[b],
            ).wait()

    def accumulate(b):
        col_e = lax.iota(jnp.int32, 16) * 2
        col_o = col_e + 1

        @pl.loop(0, C)
        def _bag(i):
            r0 = i * L
            acc = rows_v[b][r0, :]
            for l in range(1, L):
                acc = acc + rows_v[b][r0 + l, :]
            acc_e, acc_o = plsc.unpack(acc, format=plsc.PackFormat.INTERLEAVED)
            row = jnp.full((16,), i, dtype=jnp.int32)
            plsc.store_scatter(out_v[b], [row, col_e], acc_e)
            plsc.store_scatter(out_v[b], [row, col_o], acc_o)

    pltpu.sync_copy(idx_hbm.at[cid_of(0)], idx_v[0])
    fire_gathers(0)
    fire_idx(1, 1)

    @pl.loop(0, NCHUNK, step=2)
    def _outer(g):
        for b in range(2):
            k = g + b
            drain_gathers(b)

            @pl.when(k + 2 < NCHUNK)
            def _():
                fire_idx(k + 2, b)

            @pl.when(k + 1 < NCHUNK)
            def _():
                wait_idx(1 - b)
                fire_gathers(1 - b)

            @pl.when(k >= 2)
            def _():
                pltpu.make_async_copy(out_v[b], out_dst(k - 2), osem[b]).wait()

            accumulate(b)
            pltpu.async_copy(out_v[b], out_dst(k), osem[b])

    pltpu.make_async_copy(out_v[0], out_dst(NCHUNK - 2), osem[0]).wait()
    pltpu.make_async_copy(out_v[1], out_dst(NCHUNK - 1), osem[1]).wait()


@jax.jit
def _sc_lookup(idx_chunks, tab_flat):
    mesh = plsc.VectorSubcoreMesh(core_axis_name="c", subcore_axis_name="s")
    return pl.kernel(
        _sc_body,
        out_type=jax.ShapeDtypeStruct((B, FH, D), jnp.float32),
        mesh=mesh,
        scratch_types=[
            pltpu.VMEM((NDMA, 128), jnp.int32),
            pltpu.VMEM((NDMA, 128), jnp.int32),
            pltpu.VMEM((ROWS, D), jnp.bfloat16),
            pltpu.VMEM((ROWS, D), jnp.bfloat16),
            pltpu.VMEM((C, D), jnp.float32),
            pltpu.VMEM((C, D), jnp.float32),
            pltpu.SemaphoreType.DMA,
            pltpu.SemaphoreType.DMA,
            pltpu.SemaphoreType.DMA,
            pltpu.SemaphoreType.DMA,
            pltpu.SemaphoreType.DMA,
            pltpu.SemaphoreType.DMA,
        ],
        compiler_params=pltpu.CompilerParams(
            use_tc_tiling_on_sc=False, needs_layout_passes=False
        ),
    )(idx_chunks, tab_flat)


def kernel(indices, tables):
    idx0 = indices.astype(jnp.int32)
    p = idx0 % VBLK
    fl = jnp.arange(F, dtype=jnp.int32)[:, None, None] % FH
    idx = (
        (fl * TPF + idx0 // VBLK) * VBLK + (p % 1024) * 4 + p // 1024
    )
    idx128 = jax.lax.optimization_barrier(idx.reshape(F * B * L // 128, 128))
    idx_chunks = idx128.reshape(
        2, FH * B * L // (NDMA * 128), NDMA, 128
    )
    tabT = jnp.transpose(tables, (0, 2, 1))
    halves = []
    for h in range(2):
        tab_h = _tc_flatten(tabT, h * FH).reshape(FH * VP, D)
        halves.append(_sc_lookup(idx_chunks[h], tab_h))
    return jnp.concatenate(halves, axis=1)


# speedup vs baseline: 1.6554x; 1.6548x over previous
import jax
import jax.numpy as jnp
from jax import lax
from jax.experimental import pallas as pl
from jax.experimental.pallas import tpu as pltpu
from jax.experimental.pallas import tpu_sc as plsc

F = 26
V = 100000
D = 32
B = 4096
L = 20
VP = 102400
RPF = VP * D // 128

NC = 2
NS = 16
NW = NC * NS
BPW = B // NW
C = 64
S = BPW // C
ROWS = C * L
NDMA = ROWS // 128
FH = F // 2
NCHUNK = FH * S

VBLK = 4096
TPF = 25


def _tp_body(x_ref, o_ref):
    x = x_ref[0]
    xx = jnp.concatenate(
        [x[:, q * 1024:(q + 1) * 1024] for q in range(4)], axis=0
    )
    o_ref[0] = jnp.swapaxes(xx, 0, 1)


def _tc_flatten(tabT, f0):
    return pl.pallas_call(
        _tp_body,
        grid=(FH, TPF),
        in_specs=[pl.BlockSpec((1, D, VBLK), lambda f, vb: (f0 + f, 0, vb))],
        out_specs=pl.BlockSpec(
            (1, VBLK // 4, 128), lambda f, vb: (f, vb, 0)
        ),
        out_shape=jax.ShapeDtypeStruct((FH, RPF, 128), jnp.float32),
    )(tabT)


def _sc_body(idx_hbm, tab_hbm, out_hbm,
             idx_v0, idx_v1, rows_v0, rows_v1, out_v0, out_v1,
             isem0, isem1, gsem0, gsem1, osem0, osem1):
    idx_v = [idx_v0, idx_v1]
    rows_v = [rows_v0, rows_v1]
    out_v = [out_v0, out_v1]
    isem = [isem0, isem1]
    gsem = [gsem0, gsem1]
    osem = [osem0, osem1]

    wid = lax.axis_index("s") * NC + lax.axis_index("c")

    def cid_of(k):
        return (k // S) * (B // C) + wid * S + (k % S)

    def out_dst(k):
        b0 = wid * BPW + (k % S) * C
        return out_hbm.at[pl.ds(b0, C), k // S]

    def fire_idx(k, b):
        pltpu.async_copy(idx_hbm.at[cid_of(k)], idx_v[b], isem[b])

    def wait_idx(b):
        pltpu.make_async_copy(idx_hbm.at[0], idx_v[b], isem[b]).wait()

    def fire_gathers(b):
        for j in range(NDMA):
            pltpu.async_copy(
                tab_hbm.at[idx_v[b].at[j]],
                rows_v[b].at[pl.ds(j * 128, 128)],
                gsem[b],
            )

    def drain_gathers(b):
        for j in range(NDMA):
            pltpu.make_async_copy(
                tab_hbm.at[idx_v[b].at[j]],
                rows_v[b].at[pl.ds(j * 128, 128)],
                gsem[b],
            ).wait()

    def accumulate(b):
        @pl.loop(0, C)
        def _bag(i):
            r0 = i * L
            acc0 = rows_v[b][r0, pl.ds(0, 16)]
            acc1 = rows_v[b][r0, pl.ds(16, 16)]
            for l in range(1, L):
                acc0 = acc0 + rows_v[b][r0 + l, pl.ds(0, 16)]
                acc1 = acc1 + rows_v[b][r0 + l, pl.ds(16, 16)]
            out_v[b][i, pl.ds(0, 16)] = acc0
            out_v[b][i, pl.ds(16, 16)] = acc1

    pltpu.sync_copy(idx_hbm.at[cid_of(0)], idx_v[0])
    fire_gathers(0)
    fire_idx(1, 1)

    @pl.loop(0, NCHUNK, step=2)
    def _outer(g):
        for b in range(2):
            k = g + b
            drain_gathers(b)

            @pl.when(k + 2 < NCHUNK)
            def _():
                fire_idx(k + 2, b)

            @pl.when(k + 1 < NCHUNK)
            def _():
                wait_idx(1 - b)
                fire_gathers(1 - b)

            @pl.when(k >= 2)
            def _():
                pltpu.make_async_copy(out_v[b], out_dst(k - 2), osem[b]).wait()

            accumulate(b)
            pltpu.async_copy(out_v[b], out_dst(k), osem[b])

    pltpu.make_async_copy(out_v[0], out_dst(NCHUNK - 2), osem[0]).wait()
    pltpu.make_async_copy(out_v[1], out_dst(NCHUNK - 1), osem[1]).wait()


@jax.jit
def _sc_lookup(idx_chunks, tab_flat):
    mesh = plsc.VectorSubcoreMesh(core_axis_name="c", subcore_axis_name="s")
    return pl.kernel(
        _sc_body,
        out_type=jax.ShapeDtypeStruct((B, FH, D), jnp.float32),
        mesh=mesh,
        scratch_types=[
            pltpu.VMEM((NDMA, 128), jnp.int32),
            pltpu.VMEM((NDMA, 128), jnp.int32),
            pltpu.VMEM((ROWS, D), jnp.float32),
            pltpu.VMEM((ROWS, D), jnp.float32),
            pltpu.VMEM((C, D), jnp.float32),
            pltpu.VMEM((C, D), jnp.float32),
            pltpu.SemaphoreType.DMA,
            pltpu.SemaphoreType.DMA,
            pltpu.SemaphoreType.DMA,
            pltpu.SemaphoreType.DMA,
            pltpu.SemaphoreType.DMA,
            pltpu.SemaphoreType.DMA,
        ],
        compiler_params=pltpu.CompilerParams(use_tc_tiling_on_sc=False),
    )(idx_chunks, tab_flat)


def kernel(indices, tables):
    idx0 = indices.astype(jnp.int32)
    p = idx0 % VBLK
    fl = jnp.arange(F, dtype=jnp.int32)[:, None, None] % FH
    idx = (
        (fl * TPF + idx0 // VBLK) * VBLK + (p % 1024) * 4 + p // 1024
    )
    idx128 = jax.lax.optimization_barrier(idx.reshape(F * B * L // 128, 128))
    idx_chunks = idx128.reshape(
        2, FH * B * L // (NDMA * 128), NDMA, 128
    )
    tabT = jnp.transpose(tables, (0, 2, 1))
    halves = []
    for h in range(2):
        tab_h = _tc_flatten(tabT, h * FH).reshape(FH * VP, D)
        halves.append(_sc_lookup(idx_chunks[h], tab_h))
    return jnp.concatenate(halves, axis=1)


# speedup vs baseline: 2.0026x; 1.2098x over previous
import jax
import jax.numpy as jnp
from jax import lax
from jax.experimental import pallas as pl
from jax.experimental.pallas import tpu as pltpu
from jax.experimental.pallas import tpu_sc as plsc

F = 26
V = 100000
D = 32
B = 4096
L = 20
VP = 106496
RPF = VP * D // 128

NC = 2
NS = 16
NW = NC * NS
BPW = B // NW
C = 64
S = BPW // C
ROWS = C * L
NDMA = ROWS // 128
FH = F // 2
NCHUNK = FH * S

VBLK = 8192
TPF = 13


def _tp_body(x_ref, o_ref):
    x = x_ref[0]
    xx = jnp.concatenate(
        [x[:, q * (VBLK // 4):(q + 1) * (VBLK // 4)] for q in range(4)], axis=0
    )
    o_ref[0] = jnp.swapaxes(xx, 0, 1)


def _tc_flatten(tabT, f0):
    return pl.pallas_call(
        _tp_body,
        grid=(FH, TPF),
        in_specs=[pl.BlockSpec((1, D, VBLK), lambda f, vb: (f0 + f, 0, vb))],
        out_specs=pl.BlockSpec(
            (1, VBLK // 4, 128), lambda f, vb: (f, vb, 0)
        ),
        out_shape=jax.ShapeDtypeStruct((FH, RPF, 128), jnp.float32),
    )(tabT)


def _sc_body(idx_hbm, tab_hbm, out_hbm,
             idx_v0, idx_v1, rows_v0, rows_v1, out_v0, out_v1,
             isem0, isem1, gsem0, gsem1, osem0, osem1):
    idx_v = [idx_v0, idx_v1]
    rows_v = [rows_v0, rows_v1]
    out_v = [out_v0, out_v1]
    isem = [isem0, isem1]
    gsem = [gsem0, gsem1]
    osem = [osem0, osem1]

    wid = lax.axis_index("s") * NC + lax.axis_index("c")

    def cid_of(k):
        return (k // S) * (B // C) + wid * S + (k % S)

    def out_dst(k):
        b0 = wid * BPW + (k % S) * C
        return out_hbm.at[pl.ds(b0, C), k // S]

    def fire_idx(k, b):
        pltpu.async_copy(idx_hbm.at[cid_of(k)], idx_v[b], isem[b])

    def wait_idx(b):
        pltpu.make_async_copy(idx_hbm.at[0], idx_v[b], isem[b]).wait()

    def fire_gathers(b):
        for j in range(NDMA):
            pltpu.async_copy(
                tab_hbm.at[idx_v[b].at[j]],
                rows_v[b].at[pl.ds(j * 128, 128)],
                gsem[b],
            )

    def drain_gathers(b):
        for j in range(NDMA):
            pltpu.make_async_copy(
                tab_hbm.at[idx_v[b].at[j]],
                rows_v[b].at[pl.ds(j * 128, 128)],
                gsem[b],
            ).wait()

    def accumulate(b):
        @pl.loop(0, C)
        def _bag(i):
            r0 = i * L
            acc0 = rows_v[b][r0, pl.ds(0, 16)]
            acc1 = rows_v[b][r0, pl.ds(16, 16)]
            for l in range(1, L):
                acc0 = acc0 + rows_v[b][r0 + l, pl.ds(0, 16)]
                acc1 = acc1 + rows_v[b][r0 + l, pl.ds(16, 16)]
            out_v[b][i, pl.ds(0, 16)] = acc0
            out_v[b][i, pl.ds(16, 16)] = acc1

    pltpu.sync_copy(idx_hbm.at[cid_of(0)], idx_v[0])
    fire_gathers(0)
    fire_idx(1, 1)

    @pl.loop(0, NCHUNK, step=2)
    def _outer(g):
        for b in range(2):
            k = g + b
            drain_gathers(b)

            @pl.when(k + 2 < NCHUNK)
            def _():
                fire_idx(k + 2, b)

            @pl.when(k + 1 < NCHUNK)
            def _():
                wait_idx(1 - b)
                fire_gathers(1 - b)

            @pl.when(k >= 2)
            def _():
                pltpu.make_async_copy(out_v[b], out_dst(k - 2), osem[b]).wait()

            accumulate(b)
            pltpu.async_copy(out_v[b], out_dst(k), osem[b])

    pltpu.make_async_copy(out_v[0], out_dst(NCHUNK - 2), osem[0]).wait()
    pltpu.make_async_copy(out_v[1], out_dst(NCHUNK - 1), osem[1]).wait()


@jax.jit
def _sc_lookup(idx_chunks, tab_flat):
    mesh = plsc.VectorSubcoreMesh(core_axis_name="c", subcore_axis_name="s")
    return pl.kernel(
        _sc_body,
        out_type=jax.ShapeDtypeStruct((B, FH, D), jnp.float32),
        mesh=mesh,
        scratch_types=[
            pltpu.VMEM((NDMA, 128), jnp.int32),
            pltpu.VMEM((NDMA, 128), jnp.int32),
            pltpu.VMEM((ROWS, D), jnp.float32),
            pltpu.VMEM((ROWS, D), jnp.float32),
            pltpu.VMEM((C, D), jnp.float32),
            pltpu.VMEM((C, D), jnp.float32),
            pltpu.SemaphoreType.DMA,
            pltpu.SemaphoreType.DMA,
            pltpu.SemaphoreType.DMA,
            pltpu.SemaphoreType.DMA,
            pltpu.SemaphoreType.DMA,
            pltpu.SemaphoreType.DMA,
        ],
        compiler_params=pltpu.CompilerParams(use_tc_tiling_on_sc=False),
    )(idx_chunks, tab_flat)


def kernel(indices, tables):
    idx0 = indices.astype(jnp.int32)
    p = idx0 % VBLK
    fl = jnp.arange(F, dtype=jnp.int32)[:, None, None] % FH
    idx = (
        (fl * TPF + idx0 // VBLK) * VBLK
        + (p % (VBLK // 4)) * 4
        + p // (VBLK // 4)
    )
    idx128 = jax.lax.optimization_barrier(idx.reshape(F * B * L // 128, 128))
    idx_chunks = idx128.reshape(
        2, FH * B * L // (NDMA * 128), NDMA, 128
    )
    tabT = jnp.transpose(tables, (0, 2, 1))
    halves = []
    for h in range(2):
        tab_h = _tc_flatten(tabT, h * FH).reshape(FH * VP, D)
        halves.append(_sc_lookup(idx_chunks[h], tab_h))
    return jnp.concatenate(halves, axis=1)


# speedup vs baseline: 2.2491x; 1.1231x over previous
import jax
import jax.numpy as jnp
from jax import lax
from jax.experimental import pallas as pl
from jax.experimental.pallas import tpu as pltpu
from jax.experimental.pallas import tpu_sc as plsc

F = 26
V = 100000
D = 32
B = 4096
L = 20
VP = 114688
RPF = VP * D // 128

NC = 2
NS = 16
NW = NC * NS
BPW = B // NW
C = 64
S = BPW // C
ROWS = C * L
NDMA = ROWS // 128
FH = F // 2
NCHUNK = FH * S

VBLK = 16384
TPF = 7


def _tp_body(x_ref, o_ref):
    x = x_ref[0]
    xx = jnp.concatenate(
        [x[:, q * (VBLK // 4):(q + 1) * (VBLK // 4)] for q in range(4)], axis=0
    )
    o_ref[0] = jnp.swapaxes(xx, 0, 1)


def _tc_flatten(tabT, f0):
    return pl.pallas_call(
        _tp_body,
        grid=(FH, TPF),
        in_specs=[pl.BlockSpec((1, D, VBLK), lambda f, vb: (f0 + f, 0, vb))],
        out_specs=pl.BlockSpec(
            (1, VBLK // 4, 128), lambda f, vb: (f, vb, 0)
        ),
        out_shape=jax.ShapeDtypeStruct((FH, RPF, 128), jnp.float32),
    )(tabT)


def _sc_body(idx_hbm, tab_hbm, out_hbm,
             idx_v0, idx_v1, rows_v0, rows_v1, out_v0, out_v1,
             isem0, isem1, gsem0, gsem1, osem0, osem1):
    idx_v = [idx_v0, idx_v1]
    rows_v = [rows_v0, rows_v1]
    out_v = [out_v0, out_v1]
    isem = [isem0, isem1]
    gsem = [gsem0, gsem1]
    osem = [osem0, osem1]

    wid = lax.axis_index("s") * NC + lax.axis_index("c")

    def cid_of(k):
        return (k // S) * (B // C) + wid * S + (k % S)

    def out_dst(k):
        b0 = wid * BPW + (k % S) * C
        return out_hbm.at[pl.ds(b0, C), k // S]

    def fire_idx(k, b):
        pltpu.async_copy(idx_hbm.at[cid_of(k)], idx_v[b], isem[b])

    def wait_idx(b):
        pltpu.make_async_copy(idx_hbm.at[0], idx_v[b], isem[b]).wait()

    def fire_gathers(b):
        for j in range(NDMA):
            pltpu.async_copy(
                tab_hbm.at[idx_v[b].at[j]],
                rows_v[b].at[pl.ds(j * 128, 128)],
                gsem[b],
            )

    def drain_gathers(b):
        for j in range(NDMA):
            pltpu.make_async_copy(
                tab_hbm.at[idx_v[b].at[j]],
                rows_v[b].at[pl.ds(j * 128, 128)],
                gsem[b],
            ).wait()

    def accumulate(b):
        @pl.loop(0, C)
        def _bag(i):
            r0 = i * L
            acc0 = rows_v[b][r0, pl.ds(0, 16)]
            acc1 = rows_v[b][r0, pl.ds(16, 16)]
            for l in range(1, L):
                acc0 = acc0 + rows_v[b][r0 + l, pl.ds(0, 16)]
                acc1 = acc1 + rows_v[b][r0 + l, pl.ds(16, 16)]
            out_v[b][i, pl.ds(0, 16)] = acc0
            out_v[b][i, pl.ds(16, 16)] = acc1

    pltpu.sync_copy(idx_hbm.at[cid_of(0)], idx_v[0])
    fire_gathers(0)
    fire_idx(1, 1)

    @pl.loop(0, NCHUNK, step=2)
    def _outer(g):
        for b in range(2):
            k = g + b
            drain_gathers(b)

            @pl.when(k + 2 < NCHUNK)
            def _():
                fire_idx(k + 2, b)

            @pl.when(k + 1 < NCHUNK)
            def _():
                wait_idx(1 - b)
                fire_gathers(1 - b)

            @pl.when(k >= 2)
            def _():
                pltpu.make_async_copy(out_v[b], out_dst(k - 2), osem[b]).wait()

            accumulate(b)
            pltpu.async_copy(out_v[b], out_dst(k), osem[b])

    pltpu.make_async_copy(out_v[0], out_dst(NCHUNK - 2), osem[0]).wait()
    pltpu.make_async_copy(out_v[1], out_dst(NCHUNK - 1), osem[1]).wait()


@jax.jit
def _sc_lookup(idx_chunks, tab_flat):
    mesh = plsc.VectorSubcoreMesh(core_axis_name="c", subcore_axis_name="s")
    return pl.kernel(
        _sc_body,
        out_type=jax.ShapeDtypeStruct((B, FH, D), jnp.float32),
        mesh=mesh,
        scratch_types=[
            pltpu.VMEM((NDMA, 128), jnp.int32),
            pltpu.VMEM((NDMA, 128), jnp.int32),
            pltpu.VMEM((ROWS, D), jnp.float32),
            pltpu.VMEM((ROWS, D), jnp.float32),
            pltpu.VMEM((C, D), jnp.float32),
            pltpu.VMEM((C, D), jnp.float32),
            pltpu.SemaphoreType.DMA,
            pltpu.SemaphoreType.DMA,
            pltpu.SemaphoreType.DMA,
            pltpu.SemaphoreType.DMA,
            pltpu.SemaphoreType.DMA,
            pltpu.SemaphoreType.DMA,
        ],
        compiler_params=pltpu.CompilerParams(use_tc_tiling_on_sc=False),
    )(idx_chunks, tab_flat)


def kernel(indices, tables):
    idx0 = indices.astype(jnp.int32)
    p = idx0 % VBLK
    fl = jnp.arange(F, dtype=jnp.int32)[:, None, None] % FH
    idx = (
        (fl * TPF + idx0 // VBLK) * VBLK
        + (p % (VBLK // 4)) * 4
        + p // (VBLK // 4)
    )
    idx128 = jax.lax.optimization_barrier(idx.reshape(F * B * L // 128, 128))
    idx_chunks = idx128.reshape(
        2, FH * B * L // (NDMA * 128), NDMA, 128
    )
    tabT = jnp.transpose(tables, (0, 2, 1))
    halves = []
    for h in range(2):
        tab_h = _tc_flatten(tabT, h * FH).reshape(FH * VP, D)
        halves.append(_sc_lookup(idx_chunks[h], tab_h))
    return jnp.concatenate(halves, axis=1)


# speedup vs baseline: 2.2558x; 1.0030x over previous
import jax
import jax.numpy as jnp
from jax import lax
from jax.experimental import pallas as pl
from jax.experimental.pallas import tpu as pltpu
from jax.experimental.pallas import tpu_sc as plsc

F = 26
V = 100000
D = 32
B = 4096
L = 20
VP = 131072
RPF = VP * D // 128

NC = 2
NS = 16
NW = NC * NS
BPW = B // NW
C = 64
S = BPW // C
ROWS = C * L
NDMA = ROWS // 128
FH = F // 2
NCHUNK = FH * S

VBLK = 32768
TPF = 4


def _tp_body(x_ref, o_ref):
    x = x_ref[0]
    xx = jnp.concatenate(
        [x[:, q * (VBLK // 4):(q + 1) * (VBLK // 4)] for q in range(4)], axis=0
    )
    o_ref[0] = jnp.swapaxes(xx, 0, 1)


def _tc_flatten(tabT, f0):
    return pl.pallas_call(
        _tp_body,
        grid=(FH, TPF),
        in_specs=[pl.BlockSpec((1, D, VBLK), lambda f, vb: (f0 + f, 0, vb))],
        out_specs=pl.BlockSpec(
            (1, VBLK // 4, 128), lambda f, vb: (f, vb, 0)
        ),
        out_shape=jax.ShapeDtypeStruct((FH, RPF, 128), jnp.float32),
    )(tabT)


def _sc_body(idx_hbm, tab_hbm, out_hbm,
             idx_v0, idx_v1, rows_v0, rows_v1, out_v0, out_v1,
             isem0, isem1, gsem0, gsem1, osem0, osem1):
    idx_v = [idx_v0, idx_v1]
    rows_v = [rows_v0, rows_v1]
    out_v = [out_v0, out_v1]
    isem = [isem0, isem1]
    gsem = [gsem0, gsem1]
    osem = [osem0, osem1]

    wid = lax.axis_index("s") * NC + lax.axis_index("c")

    def cid_of(k):
        return (k // S) * (B // C) + wid * S + (k % S)

    def out_dst(k):
        b0 = wid * BPW + (k % S) * C
        return out_hbm.at[pl.ds(b0, C), k // S]

    def fire_idx(k, b):
        pltpu.async_copy(idx_hbm.at[cid_of(k)], idx_v[b], isem[b])

    def wait_idx(b):
        pltpu.make_async_copy(idx_hbm.at[0], idx_v[b], isem[b]).wait()

    def fire_gathers(b):
        for j in range(NDMA):
            pltpu.async_copy(
                tab_hbm.at[idx_v[b].at[j]],
                rows_v[b].at[pl.ds(j * 128, 128)],
                gsem[b],
            )

    def drain_gathers(b):
        for j in range(NDMA):
            pltpu.make_async_copy(
                tab_hbm.at[idx_v[b].at[j]],
                rows_v[b].at[pl.ds(j * 128, 128)],
                gsem[b],
            ).wait()

    def accumulate(b):
        @pl.loop(0, C)
        def _bag(i):
            r0 = i * L
            acc0 = rows_v[b][r0, pl.ds(0, 16)]
            acc1 = rows_v[b][r0, pl.ds(16, 16)]
            for l in range(1, L):
                acc0 = acc0 + rows_v[b][r0 + l, pl.ds(0, 16)]
                acc1 = acc1 + rows_v[b][r0 + l, pl.ds(16, 16)]
            out_v[b][i, pl.ds(0, 16)] = acc0
            out_v[b][i, pl.ds(16, 16)] = acc1

    pltpu.sync_copy(idx_hbm.at[cid_of(0)], idx_v[0])
    fire_gathers(0)
    fire_idx(1, 1)

    @pl.loop(0, NCHUNK, step=2)
    def _outer(g):
        for b in range(2):
            k = g + b
            drain_gathers(b)

            @pl.when(k + 2 < NCHUNK)
            def _():
                fire_idx(k + 2, b)

            @pl.when(k + 1 < NCHUNK)
            def _():
                wait_idx(1 - b)
                fire_gathers(1 - b)

            @pl.when(k >= 2)
            def _():
                pltpu.make_async_copy(out_v[b], out_dst(k - 2), osem[b]).wait()

            accumulate(b)
            pltpu.async_copy(out_v[b], out_dst(k), osem[b])

    pltpu.make_async_copy(out_v[0], out_dst(NCHUNK - 2), osem[0]).wait()
    pltpu.make_async_copy(out_v[1], out_dst(NCHUNK - 1), osem[1]).wait()


@jax.jit
def _sc_lookup(idx_chunks, tab_flat):
    mesh = plsc.VectorSubcoreMesh(core_axis_name="c", subcore_axis_name="s")
    return pl.kernel(
        _sc_body,
        out_type=jax.ShapeDtypeStruct((B, FH, D), jnp.float32),
        mesh=mesh,
        scratch_types=[
            pltpu.VMEM((NDMA, 128), jnp.int32),
            pltpu.VMEM((NDMA, 128), jnp.int32),
            pltpu.VMEM((ROWS, D), jnp.float32),
            pltpu.VMEM((ROWS, D), jnp.float32),
            pltpu.VMEM((C, D), jnp.float32),
            pltpu.VMEM((C, D), jnp.float32),
            pltpu.SemaphoreType.DMA,
            pltpu.SemaphoreType.DMA,
            pltpu.SemaphoreType.DMA,
            pltpu.SemaphoreType.DMA,
            pltpu.SemaphoreType.DMA,
            pltpu.SemaphoreType.DMA,
        ],
        compiler_params=pltpu.CompilerParams(use_tc_tiling_on_sc=False),
    )(idx_chunks, tab_flat)


def kernel(indices, tables):
    idx0 = indices.astype(jnp.int32)
    p = idx0 % VBLK
    fl = jnp.arange(F, dtype=jnp.int32)[:, None, None] % FH
    idx = (
        (fl * TPF + idx0 // VBLK) * VBLK
        + (p % (VBLK // 4)) * 4
        + p // (VBLK // 4)
    )
    idx128 = jax.lax.optimization_barrier(idx.reshape(F * B * L // 128, 128))
    idx_chunks = idx128.reshape(
        2, FH * B * L // (NDMA * 128), NDMA, 128
    )
    tabT = jnp.transpose(tables, (0, 2, 1))
    halves = []
    for h in range(2):
        tab_h = _tc_flatten(tabT, h * FH).reshape(FH * VP, D)
        halves.append(_sc_lookup(idx_chunks[h], tab_h))
    return jnp.concatenate(halves, axis=1)
